# baseline (device time: 17240 ns/iter reference)
import jax
import jax.numpy as jnp
from jax import lax
from jax.experimental import pallas as pl
from jax.experimental.pallas import tpu as pltpu

N_DEV = 16
BM = 256


def kernel(x):
    m, n = x.shape
    assert m % BM == 0
    g = m // BM
    n_iters = 2 * g

    def body(x_ref, x_hbm, out_ref, halo_ref, carry_ref, send_sems, recv_sems):
        t = pl.program_id(0)
        my = lax.axis_index("i")
        left = lax.rem(my - 1 + N_DEV, N_DEV)
        right = lax.rem(my + 1, N_DEV)

        send_right = pltpu.make_async_remote_copy(
            src_ref=x_hbm.at[pl.ds(m - 1, 1)],
            dst_ref=halo_ref.at[0],
            send_sem=send_sems.at[0],
            recv_sem=recv_sems.at[0],
            device_id=(right,),
            device_id_type=pl.DeviceIdType.MESH,
        )
        send_left = pltpu.make_async_remote_copy(
            src_ref=x_hbm.at[pl.ds(0, 1)],
            dst_ref=halo_ref.at[1],
            send_sem=send_sems.at[1],
            recv_sem=recv_sems.at[1],
            device_id=(left,),
            device_id_type=pl.DeviceIdType.MESH,
        )

        @pl.when(t == 0)
        def _():
            barrier_sem = pltpu.get_barrier_semaphore()
            pl.semaphore_signal(
                barrier_sem, inc=1, device_id=(left,),
                device_id_type=pl.DeviceIdType.MESH,
            )
            pl.semaphore_signal(
                barrier_sem, inc=1, device_id=(right,),
                device_id_type=pl.DeviceIdType.MESH,
            )
            pl.semaphore_wait(barrier_sem, 2)
            send_right.start()
            send_left.start()

        even = lax.rem(t, 2) == 0

        @pl.when(even)
        def _():
            xv = x_ref[...]
            out_ref[1 : BM - 1, :] = (
                0.25 * xv[0 : BM - 2] + 0.5 * xv[1 : BM - 1] + 0.25 * xv[2:BM]
            ).astype(jnp.bfloat16)

            @pl.when(t == 0)
            def _():
                send_right.wait_recv()
                row0 = 0.25 * halo_ref[0, :, :] + 0.5 * xv[0:1] + 0.25 * xv[1:2]
                out_ref[0:1, :] = jnp.where(my == 0, xv[0:1], row0).astype(
                    jnp.bfloat16
                )

            @pl.when(t != 0)
            def _():
                row0 = 0.25 * carry_ref[1:2, :] + 0.5 * xv[0:1] + 0.25 * xv[1:2]
                out_ref[0:1, :] = row0.astype(jnp.bfloat16)

            carry_ref[0:1, :] = xv[BM - 2 : BM - 1]
            carry_ref[1:2, :] = xv[BM - 1 : BM]

        @pl.when(jnp.logical_not(even))
        def _():
            @pl.when(t != n_iters - 1)
            def _():
                last = (
                    0.25 * carry_ref[0:1, :]
                    + 0.5 * carry_ref[1:2, :]
                    + 0.25 * x_ref[0:1, :]
                )
                out_ref[BM - 1 : BM, :] = last.astype(jnp.bfloat16)

            @pl.when(t == n_iters - 1)
            def _():
                send_left.wait_recv()
                last = (
                    0.25 * carry_ref[0:1, :]
                    + 0.5 * carry_ref[1:2, :]
                    + 0.25 * halo_ref[1, :, :]
                )
                out_ref[BM - 1 : BM, :] = jnp.where(
                    my == N_DEV - 1, carry_ref[1:2, :], last
                ).astype(jnp.bfloat16)
                send_right.wait_send()
                send_left.wait_send()

    return pl.pallas_call(
        body,
        grid=(n_iters,),
        out_shape=jax.ShapeDtypeStruct((m, n), jnp.bfloat16),
        in_specs=[
            pl.BlockSpec(
                (BM, n),
                lambda t: (jnp.minimum((t + 1) // 2, g - 1), 0),
                memory_space=pltpu.VMEM,
            ),
            pl.BlockSpec(memory_space=pl.ANY),
        ],
        out_specs=pl.BlockSpec(
            (BM, n), lambda t: (t // 2, 0), memory_space=pltpu.VMEM
        ),
        scratch_shapes=[
            pltpu.VMEM((2, 1, n), x.dtype),
            pltpu.VMEM((2, n), x.dtype),
            pltpu.SemaphoreType.DMA((2,)),
            pltpu.SemaphoreType.DMA((2,)),
        ],
        compiler_params=pltpu.CompilerParams(
            collective_id=0, dimension_semantics=("arbitrary",)
        ),
    )(x, x)


# device time: 16907 ns/iter; 1.0197x vs baseline; 1.0197x over previous
import jax
import jax.numpy as jnp
from jax import lax
from jax.experimental import pallas as pl
from jax.experimental.pallas import tpu as pltpu

N_DEV = 16
BM = 256


def kernel(x):
    m, n = x.shape
    assert m % BM == 0
    g = m // BM

    def body(
        x_ref, x_hbm, out_ref, halo_ref, carry_ref, next_ref,
        send_sems, recv_sems, copy_sem,
    ):
        k = pl.program_id(0)
        my = lax.axis_index("i")
        left = lax.rem(my - 1 + N_DEV, N_DEV)
        right = lax.rem(my + 1, N_DEV)

        send_right = pltpu.make_async_remote_copy(
            src_ref=x_hbm.at[pl.ds(m - 1, 1)],
            dst_ref=halo_ref.at[0],
            send_sem=send_sems.at[0],
            recv_sem=recv_sems.at[0],
            device_id=(right,),
            device_id_type=pl.DeviceIdType.MESH,
        )
        send_left = pltpu.make_async_remote_copy(
            src_ref=x_hbm.at[pl.ds(0, 1)],
            dst_ref=halo_ref.at[1],
            send_sem=send_sems.at[1],
            recv_sem=recv_sems.at[1],
            device_id=(left,),
            device_id_type=pl.DeviceIdType.MESH,
        )

        @pl.when(k == 0)
        def _():
            barrier_sem = pltpu.get_barrier_semaphore()
            pl.semaphore_signal(
                barrier_sem, inc=1, device_id=(left,),
                device_id_type=pl.DeviceIdType.MESH,
            )
            pl.semaphore_signal(
                barrier_sem, inc=1, device_id=(right,),
                device_id_type=pl.DeviceIdType.MESH,
            )
            pl.semaphore_wait(barrier_sem, 2)
            send_right.start()
            send_left.start()

        nxt = pltpu.make_async_copy(
            x_hbm.at[pl.ds(jnp.minimum((k + 1) * BM, m - 1), 1)],
            next_ref,
            copy_sem,
        )
        nxt.start()

        xv = x_ref[...]
        out_ref[1 : BM - 1, :] = (
            0.25 * xv[0 : BM - 2] + 0.5 * xv[1 : BM - 1] + 0.25 * xv[2:BM]
        ).astype(jnp.bfloat16)

        @pl.when(k == 0)
        def _():
            send_right.wait_recv()
            row0 = 0.25 * halo_ref[0, :, :] + 0.5 * xv[0:1] + 0.25 * xv[1:2]
            out_ref[0:1, :] = jnp.where(my == 0, xv[0:1], row0).astype(
                jnp.bfloat16
            )

        @pl.when(k != 0)
        def _():
            row0 = 0.25 * carry_ref[0:1, :] + 0.5 * xv[0:1] + 0.25 * xv[1:2]
            out_ref[0:1, :] = row0.astype(jnp.bfloat16)

        nxt.wait()

        @pl.when(k != g - 1)
        def _():
            last = (
                0.25 * xv[BM - 2 : BM - 1]
                + 0.5 * xv[BM - 1 : BM]
                + 0.25 * next_ref[0:1, :]
            )
            out_ref[BM - 1 : BM, :] = last.astype(jnp.bfloat16)

        @pl.when(k == g - 1)
        def _():
            send_left.wait_recv()
            last = (
                0.25 * xv[BM - 2 : BM - 1]
                + 0.5 * xv[BM - 1 : BM]
                + 0.25 * halo_ref[1, :, :]
            )
            out_ref[BM - 1 : BM, :] = jnp.where(
                my == N_DEV - 1, xv[BM - 1 : BM], last
            ).astype(jnp.bfloat16)
            send_right.wait_send()
            send_left.wait_send()

        carry_ref[0:1, :] = xv[BM - 1 : BM]

    return pl.pallas_call(
        body,
        grid=(g,),
        out_shape=jax.ShapeDtypeStruct((m, n), jnp.bfloat16),
        in_specs=[
            pl.BlockSpec((BM, n), lambda k: (k, 0), memory_space=pltpu.VMEM),
            pl.BlockSpec(memory_space=pl.ANY),
        ],
        out_specs=pl.BlockSpec((BM, n), lambda k: (k, 0), memory_space=pltpu.VMEM),
        scratch_shapes=[
            pltpu.VMEM((2, 1, n), x.dtype),
            pltpu.VMEM((1, n), x.dtype),
            pltpu.VMEM((1, n), x.dtype),
            pltpu.SemaphoreType.DMA((2,)),
            pltpu.SemaphoreType.DMA((2,)),
            pltpu.SemaphoreType.DMA,
        ],
        compiler_params=pltpu.CompilerParams(
            collective_id=0, dimension_semantics=("arbitrary",)
        ),
    )(x, x)


# device time: 15278 ns/iter; 1.1284x vs baseline; 1.1066x over previous
import jax
import jax.numpy as jnp
from jax import lax
from jax.experimental import pallas as pl
from jax.experimental.pallas import tpu as pltpu

N_DEV = 16
BM = 256


def kernel(x):
    m, n = x.shape
    assert m % BM == 0
    g = m // BM

    def body(x_hbm, out_hbm, xbuf, obuf, halo_ref,
             in_sems, out_sems, send_sems, recv_sems):
        my = lax.axis_index("i")
        left = lax.rem(my - 1 + N_DEV, N_DEV)
        right = lax.rem(my + 1, N_DEV)

        barrier_sem = pltpu.get_barrier_semaphore()
        pl.semaphore_signal(
            barrier_sem, inc=1, device_id=(left,),
            device_id_type=pl.DeviceIdType.MESH,
        )
        pl.semaphore_signal(
            barrier_sem, inc=1, device_id=(right,),
            device_id_type=pl.DeviceIdType.MESH,
        )
        pl.semaphore_wait(barrier_sem, 2)

        send_right = pltpu.make_async_remote_copy(
            src_ref=x_hbm.at[pl.ds(m - 1, 1)],
            dst_ref=halo_ref.at[0],
            send_sem=send_sems.at[0],
            recv_sem=recv_sems.at[0],
            device_id=(right,),
            device_id_type=pl.DeviceIdType.MESH,
        )
        send_left = pltpu.make_async_remote_copy(
            src_ref=x_hbm.at[pl.ds(0, 1)],
            dst_ref=halo_ref.at[1],
            send_sem=send_sems.at[1],
            recv_sem=recv_sems.at[1],
            device_id=(left,),
            device_id_type=pl.DeviceIdType.MESH,
        )
        send_right.start()
        send_left.start()

        in_copies = []
        for j in range(g):
            c = pltpu.make_async_copy(
                x_hbm.at[pl.ds(j * BM, BM)],
                xbuf.at[pl.ds(j * BM, BM)],
                in_sems.at[j],
            )
            c.start()
            in_copies.append(c)

        out_copies = []

        def flush_block(b):
            c = pltpu.make_async_copy(
                obuf.at[pl.ds(b * BM, BM)],
                out_hbm.at[pl.ds(b * BM, BM)],
                out_sems.at[b],
            )
            c.start()
            out_copies.append(c)

        in_copies[0].wait()
        for j in range(1, g):
            in_copies[j].wait()
            a = (j - 1) * BM + 1
            b = j * BM + 1
            obuf[a:b, :] = (
                0.25 * xbuf[a - 1 : b - 1]
                + 0.5 * xbuf[a:b]
                + 0.25 * xbuf[a + 1 : b + 1]
            ).astype(jnp.bfloat16)
            if j >= 2:
                flush_block(j - 1)

        a = (g - 1) * BM + 1
        obuf[a : m - 1, :] = (
            0.25 * xbuf[a - 1 : m - 2]
            + 0.5 * xbuf[a : m - 1]
            + 0.25 * xbuf[a + 1 : m]
        ).astype(jnp.bfloat16)

        send_right.wait_recv()
        first = 0.25 * halo_ref[0, :, :] + 0.5 * xbuf[0:1] + 0.25 * xbuf[1:2]
        obuf[0:1, :] = jnp.where(my == 0, xbuf[0:1], first).astype(jnp.bfloat16)
        send_left.wait_recv()
        last = (
            0.25 * xbuf[m - 2 : m - 1]
            + 0.5 * xbuf[m - 1 : m]
            + 0.25 * halo_ref[1, :, :]
        )
        obuf[m - 1 : m, :] = jnp.where(
            my == N_DEV - 1, xbuf[m - 1 : m], last
        ).astype(jnp.bfloat16)

        flush_block(0)
        flush_block(g - 1)

        for c in out_copies:
            c.wait()
        send_right.wait_send()
        send_left.wait_send()

    return pl.pallas_call(
        body,
        out_shape=jax.ShapeDtypeStruct((m, n), jnp.bfloat16),
        in_specs=[pl.BlockSpec(memory_space=pl.ANY)],
        out_specs=pl.BlockSpec(memory_space=pl.ANY),
        scratch_shapes=[
            pltpu.VMEM((m, n), x.dtype),
            pltpu.VMEM((m, n), jnp.bfloat16),
            pltpu.VMEM((2, 1, n), x.dtype),
            pltpu.SemaphoreType.DMA((g,)),
            pltpu.SemaphoreType.DMA((g,)),
            pltpu.SemaphoreType.DMA((2,)),
            pltpu.SemaphoreType.DMA((2,)),
        ],
        compiler_params=pltpu.CompilerParams(collective_id=0),
    )(x)


# device time: 14761 ns/iter; 1.1679x vs baseline; 1.0350x over previous
import jax
import jax.numpy as jnp
from jax import lax
from jax.experimental import pallas as pl
from jax.experimental.pallas import tpu as pltpu

N_DEV = 16


def kernel(x):
    m, n = x.shape

    def body(x_ref, out_ref, halo_ref, send_sems, recv_sems):
        my = lax.axis_index("i")
        left = lax.rem(my - 1 + N_DEV, N_DEV)
        right = lax.rem(my + 1, N_DEV)

        barrier_sem = pltpu.get_barrier_semaphore()
        pl.semaphore_signal(
            barrier_sem, inc=1, device_id=(left,),
            device_id_type=pl.DeviceIdType.MESH,
        )
        pl.semaphore_signal(
            barrier_sem, inc=1, device_id=(right,),
            device_id_type=pl.DeviceIdType.MESH,
        )
        pl.semaphore_wait(barrier_sem, 2)

        send_right = pltpu.make_async_remote_copy(
            src_ref=x_ref.at[pl.ds(m - 1, 1)],
            dst_ref=halo_ref.at[0],
            send_sem=send_sems.at[0],
            recv_sem=recv_sems.at[0],
            device_id=(right,),
            device_id_type=pl.DeviceIdType.MESH,
        )
        send_left = pltpu.make_async_remote_copy(
            src_ref=x_ref.at[pl.ds(0, 1)],
            dst_ref=halo_ref.at[1],
            send_sem=send_sems.at[1],
            recv_sem=recv_sems.at[1],
            device_id=(left,),
            device_id_type=pl.DeviceIdType.MESH,
        )
        send_right.start()
        send_left.start()

        xv = x_ref[...]
        out_ref[1 : m - 1, :] = (
            0.25 * xv[0 : m - 2] + 0.5 * xv[1 : m - 1] + 0.25 * xv[2:m]
        ).astype(jnp.bfloat16)

        send_right.wait()
        send_left.wait()

        top = halo_ref[0, :, :]
        bot = halo_ref[1, :, :]

        first = 0.25 * top + 0.5 * xv[0:1] + 0.25 * xv[1:2]
        out_ref[0:1, :] = jnp.where(my == 0, xv[0:1], first).astype(jnp.bfloat16)

        last = 0.25 * xv[m - 2 : m - 1] + 0.5 * xv[m - 1 : m] + 0.25 * bot
        out_ref[m - 1 : m, :] = jnp.where(
            my == N_DEV - 1, xv[m - 1 : m], last
        ).astype(jnp.bfloat16)

    return pl.pallas_call(
        body,
        out_shape=jax.ShapeDtypeStruct((m, n), jnp.bfloat16),
        in_specs=[pl.BlockSpec(memory_space=pltpu.VMEM)],
        out_specs=pl.BlockSpec(memory_space=pltpu.VMEM),
        scratch_shapes=[
            pltpu.VMEM((2, 1, n), x.dtype),
            pltpu.SemaphoreType.DMA((2,)),
            pltpu.SemaphoreType.DMA((2,)),
        ],
        compiler_params=pltpu.CompilerParams(collective_id=0),
    )(x)
